# Initial kernel scaffold; baseline (speedup 1.0000x reference)
import jax
import jax.numpy as jnp
from jax.experimental import pallas as pl


def kernel(x, edge_index, W0, b0, g0, be0, W1, b1, g1, be1):
    raise NotImplementedError("write your pallas kernel here")


# speedup vs baseline: 1.7223x; 1.7223x over previous
import functools

import jax
import jax.numpy as jnp
from jax import lax
from jax.experimental import pallas as pl
from jax.experimental.pallas import tpu as pltpu
from jax.experimental.pallas import tpu_sc as plsc

N = 10000
D = 128
E = 320000

NC = 2
NS = 16
NW = NC * NS
FPW = D // NW
LANES = 16
CH = 2000
NCH = E // CH
GPC = CH // LANES
ACC_W = FPW * N


def _mlp1_body(x_ref, w_ref, b_ref, g_ref, be_ref, o_ref):
    h = jnp.dot(x_ref[...], w_ref[...], preferred_element_type=jnp.float32,
                precision=lax.Precision.HIGHEST)
    h = h + b_ref[...]
    mu = jnp.mean(h, axis=-1, keepdims=True)
    var = jnp.mean(jnp.square(h - mu), axis=-1, keepdims=True)
    h = (h - mu) / jnp.sqrt(var + 1e-5) * g_ref[...] + be_ref[...]
    o_ref[...] = jnp.maximum(h, 0.0)


def _mlp2_body(h_ref, a_ref, wa_ref, wb_ref, b_ref, g_ref, be_ref, o_ref):
    h = jnp.dot(h_ref[...], wa_ref[...], preferred_element_type=jnp.float32,
                precision=lax.Precision.HIGHEST)
    h = h + jnp.dot(a_ref[...], wb_ref[...], preferred_element_type=jnp.float32,
                    precision=lax.Precision.HIGHEST)
    h = h + b_ref[...]
    mu = jnp.mean(h, axis=-1, keepdims=True)
    var = jnp.mean(jnp.square(h - mu), axis=-1, keepdims=True)
    h = (h - mu) / jnp.sqrt(var + 1e-5) * g_ref[...] + be_ref[...]
    o_ref[...] = jnp.maximum(h, 0.0)


_ROWS = 1000


def _mlp1(x, w, b, g, be):
    vec = pl.BlockSpec((1, D), lambda i: (0, 0))
    return pl.pallas_call(
        _mlp1_body,
        grid=(N // _ROWS,),
        in_specs=[
            pl.BlockSpec((_ROWS, D), lambda i: (i, 0)),
            pl.BlockSpec((D, D), lambda i: (0, 0)),
            vec, vec, vec,
        ],
        out_specs=pl.BlockSpec((_ROWS, D), lambda i: (i, 0)),
        out_shape=jax.ShapeDtypeStruct((N, D), jnp.float32),
    )(x, w, b.reshape(1, D), g.reshape(1, D), be.reshape(1, D))


def _mlp2(h, a, wa, wb, b, g, be):
    vec = pl.BlockSpec((1, D), lambda i: (0, 0))
    return pl.pallas_call(
        _mlp2_body,
        grid=(N // _ROWS,),
        in_specs=[
            pl.BlockSpec((_ROWS, D), lambda i: (i, 0)),
            pl.BlockSpec((_ROWS, D), lambda i: (i, 0)),
            pl.BlockSpec((D, D), lambda i: (0, 0)),
            pl.BlockSpec((D, D), lambda i: (0, 0)),
            vec, vec, vec,
        ],
        out_specs=pl.BlockSpec((_ROWS, D), lambda i: (i, 0)),
        out_shape=jax.ShapeDtypeStruct((N, D), jnp.float32),
    )(h, a, wa, wb, b.reshape(1, D), g.reshape(1, D), be.reshape(1, D))


def _gcn_sc_body(hc, src, dst, out, hcols, acc, scr,
                 sbufa, dbufa, sbufb, dbufb, sema, semb):
    wid = lax.axis_index("s") * NC + lax.axis_index("c")
    lane = jnp.arange(LANES, dtype=jnp.int32)

    pltpu.sync_copy(hc.at[wid], hcols)

    def _init(i, _):
        acc[pl.ds(i * LANES, LANES)] = jnp.full((LANES,), -1.0, jnp.float32)
        return _
    lax.fori_loop(0, ACC_W // LANES, _init, 0, unroll=4)

    def _start(c, sbuf, dbuf, sem):
        pltpu.async_copy(src.at[pl.ds(c * CH, CH)], sbuf, sem)
        pltpu.async_copy(dst.at[pl.ds(c * CH, CH)], dbuf, sem)

    def _wait(sbuf, dbuf, sem):
        pltpu.make_async_copy(src.at[pl.ds(0, CH)], sbuf, sem).wait()
        pltpu.make_async_copy(dst.at[pl.ds(0, CH)], dbuf, sem).wait()

    def _groups(sbuf, dbuf):
        def _g(g, _):
            s = sbuf[pl.ds(g * LANES, LANES)]
            d = dbuf[pl.ds(g * LANES, LANES)]
            s4 = s * FPW
            d4 = d * FPW
            plsc.store_scatter(scr, [d], lane)
            rb = plsc.load_gather(scr, [d])
            own = rb == lane
            vs = []
            for f in range(FPW):
                v = plsc.load_gather(hcols, [s4 + f])
                cur = plsc.load_gather(acc, [d4 + f])
                m = jnp.maximum(cur, v)
                plsc.store_scatter(acc, [d4 + f], m, mask=own)
                vs.append(v)
            rem = jnp.logical_not(own)

            @pl.when(jnp.any(rem))
            def _fallback():
                def _cond(carry):
                    r, it = carry
                    return jnp.logical_and(jnp.any(r), it < LANES)

                def _body(carry):
                    r, it = carry
                    plsc.store_scatter(scr, [d], lane, mask=r)
                    rb2 = plsc.load_gather(scr, [d])
                    own2 = jnp.logical_and(rb2 == lane, r)
                    for f in range(FPW):
                        cur2 = plsc.load_gather(acc, [d4 + f])
                        m2 = jnp.maximum(cur2, vs[f])
                        plsc.store_scatter(acc, [d4 + f], m2, mask=own2)
                    return jnp.logical_and(r, jnp.logical_not(own2)), it + 1

                lax.while_loop(_cond, _body,
                               (rem, jnp.int32(0)))
            return _
        lax.fori_loop(0, GPC, _g, 0)

    _start(0, sbufa, dbufa, sema)

    def _chunks(i, carry):
        ca = 2 * i
        _start(ca + 1, sbufb, dbufb, semb)
        _wait(sbufa, dbufa, sema)
        _groups(sbufa, dbufa)

        @pl.when(ca + 2 < NCH)
        def _prefetch():
            _start(ca + 2, sbufa, dbufa, sema)
        _wait(sbufb, dbufb, semb)
        _groups(sbufb, dbufb)
        return carry
    lax.fori_loop(0, NCH // 2, _chunks, 0)

    def _fix(i, _):
        sl = pl.ds(i * LANES, LANES)
        a = acc[sl]
        acc[sl] = jnp.where(a < 0, hcols[sl], a)
        return _
    lax.fori_loop(0, ACC_W // LANES, _fix, 0, unroll=4)

    pltpu.sync_copy(acc, out.at[wid])


@functools.partial(
    pl.kernel,
    mesh=plsc.VectorSubcoreMesh(core_axis_name="c", subcore_axis_name="s"),
    out_type=jax.ShapeDtypeStruct((NW, ACC_W), jnp.float32),
    compiler_params=pltpu.CompilerParams(needs_layout_passes=False),
    scratch_types=[
        pltpu.VMEM((ACC_W,), jnp.float32),
        pltpu.VMEM((ACC_W,), jnp.float32),
        pltpu.VMEM((N,), jnp.int32),
        pltpu.VMEM((CH,), jnp.int32),
        pltpu.VMEM((CH,), jnp.int32),
        pltpu.VMEM((CH,), jnp.int32),
        pltpu.VMEM((CH,), jnp.int32),
        pltpu.SemaphoreType.DMA,
        pltpu.SemaphoreType.DMA,
    ],
)
def _gcn_sc(hc, src, dst, out, hcols, acc, scr,
            sbufa, dbufa, sbufb, dbufb, sema, semb):
    _gcn_sc_body(hc, src, dst, out, hcols, acc, scr,
                 sbufa, dbufa, sbufb, dbufb, sema, semb)


def _to_blocked(h):
    return h.reshape(N, NW, FPW).transpose(1, 0, 2).reshape(NW, ACC_W)


def _from_blocked(a):
    return a.reshape(NW, N, FPW).transpose(1, 0, 2).reshape(N, D)


def kernel(x, edge_index, W0, b0, g0, be0, W1, b1, g1, be1):
    ei = edge_index.astype(jnp.int32)
    src = ei[0]
    dst = ei[1]

    h1 = _mlp1(x, W0, b0, g0, be0)
    a1 = _from_blocked(_gcn_sc(_to_blocked(h1), src, dst))
    h2 = _mlp2(h1, a1, W1[:D], W1[D:], b1, g1, be1)
    a2 = _from_blocked(_gcn_sc(_to_blocked(h2), src, dst))
    return jnp.concatenate([h2, a2], axis=1)


# speedup vs baseline: 2.2215x; 1.2898x over previous
import functools

import jax
import jax.numpy as jnp
from jax import lax
from jax.experimental import pallas as pl
from jax.experimental.pallas import tpu as pltpu
from jax.experimental.pallas import tpu_sc as plsc

N = 10000
D = 128
E = 320000

NC = 2
NS = 16
NW = NC * NS
FPW = D // NW
LANES = 16
CH = 2000
NCH = E // CH
GPC = CH // LANES


def _mlp1_body(x_ref, w_ref, b_ref, g_ref, be_ref, o_ref):
    h = jnp.dot(x_ref[...], w_ref[...], preferred_element_type=jnp.float32,
                precision=lax.Precision.HIGHEST)
    h = h + b_ref[...]
    mu = jnp.mean(h, axis=-1, keepdims=True)
    var = jnp.mean(jnp.square(h - mu), axis=-1, keepdims=True)
    h = (h - mu) / jnp.sqrt(var + 1e-5) * g_ref[...] + be_ref[...]
    o_ref[...] = jnp.maximum(h, 0.0)


def _mlp2_body(h_ref, a_ref, wa_ref, wb_ref, b_ref, g_ref, be_ref, o_ref):
    h = jnp.dot(h_ref[...], wa_ref[...], preferred_element_type=jnp.float32,
                precision=lax.Precision.HIGHEST)
    h = h + jnp.dot(a_ref[...], wb_ref[...], preferred_element_type=jnp.float32,
                    precision=lax.Precision.HIGHEST)
    h = h + b_ref[...]
    mu = jnp.mean(h, axis=-1, keepdims=True)
    var = jnp.mean(jnp.square(h - mu), axis=-1, keepdims=True)
    h = (h - mu) / jnp.sqrt(var + 1e-5) * g_ref[...] + be_ref[...]
    o_ref[...] = jnp.maximum(h, 0.0)


_ROWS = 1000


def _mlp1(x, w, b, g, be):
    vec = pl.BlockSpec((1, D), lambda i: (0, 0))
    return pl.pallas_call(
        _mlp1_body,
        grid=(N // _ROWS,),
        in_specs=[
            pl.BlockSpec((_ROWS, D), lambda i: (i, 0)),
            pl.BlockSpec((D, D), lambda i: (0, 0)),
            vec, vec, vec,
        ],
        out_specs=pl.BlockSpec((_ROWS, D), lambda i: (i, 0)),
        out_shape=jax.ShapeDtypeStruct((N, D), jnp.float32),
    )(x, w, b.reshape(1, D), g.reshape(1, D), be.reshape(1, D))


def _mlp2(h, a, wa, wb, b, g, be):
    vec = pl.BlockSpec((1, D), lambda i: (0, 0))
    return pl.pallas_call(
        _mlp2_body,
        grid=(N // _ROWS,),
        in_specs=[
            pl.BlockSpec((_ROWS, D), lambda i: (i, 0)),
            pl.BlockSpec((_ROWS, D), lambda i: (i, 0)),
            pl.BlockSpec((D, D), lambda i: (0, 0)),
            pl.BlockSpec((D, D), lambda i: (0, 0)),
            vec, vec, vec,
        ],
        out_specs=pl.BlockSpec((_ROWS, D), lambda i: (i, 0)),
        out_shape=jax.ShapeDtypeStruct((N, D), jnp.float32),
    )(h, a, wa, wb, b.reshape(1, D), g.reshape(1, D), be.reshape(1, D))


def _gcn_sc_body(hc, src, dst, out, hc0, hc1, hc2, hc3,
                 ac0, ac1, ac2, ac3, scr,
                 sbufa, dbufa, sbufb, dbufb, sema, semb):
    wid = lax.axis_index("s") * NC + lax.axis_index("c")
    lane = jnp.arange(LANES, dtype=jnp.int32)
    hcs = (hc0, hc1, hc2, hc3)
    acs = (ac0, ac1, ac2, ac3)

    for f in range(FPW):
        pltpu.sync_copy(hc.at[wid, f], hcs[f])

    def _init(i, carry):
        neg = jnp.full((LANES,), -1.0, jnp.float32)
        for f in range(FPW):
            acs[f][pl.ds(i * LANES, LANES)] = neg
        return carry
    lax.fori_loop(0, N // LANES, _init, 0, unroll=4)

    def _start(c, sbuf, dbuf, sem):
        pltpu.async_copy(src.at[pl.ds(c * CH, CH)], sbuf, sem)
        pltpu.async_copy(dst.at[pl.ds(c * CH, CH)], dbuf, sem)

    def _wait(sbuf, dbuf, sem):
        pltpu.make_async_copy(src.at[pl.ds(0, CH)], sbuf, sem).wait()
        pltpu.make_async_copy(dst.at[pl.ds(0, CH)], dbuf, sem).wait()

    def _groups(sbuf, dbuf):
        def _g(g, carry):
            s = sbuf[pl.ds(g * LANES, LANES)]
            d = dbuf[pl.ds(g * LANES, LANES)]
            plsc.store_scatter(scr, [d], lane)
            rb = plsc.load_gather(scr, [d])
            own = rb == lane
            vs = []
            for f in range(FPW):
                v = plsc.load_gather(hcs[f], [s])
                cur = plsc.load_gather(acs[f], [d])
                m = jnp.maximum(cur, v)
                plsc.store_scatter(acs[f], [d], m, mask=own)
                vs.append(v)
            rem = jnp.logical_not(own)

            @pl.when(jnp.any(rem))
            def _fallback():
                def _cond(c2):
                    r, it = c2
                    return jnp.logical_and(jnp.any(r), it < LANES)

                def _body(c2):
                    r, it = c2
                    plsc.store_scatter(scr, [d], lane, mask=r)
                    rb2 = plsc.load_gather(scr, [d])
                    own2 = jnp.logical_and(rb2 == lane, r)
                    for f in range(FPW):
                        cur2 = plsc.load_gather(acs[f], [d])
                        m2 = jnp.maximum(cur2, vs[f])
                        plsc.store_scatter(acs[f], [d], m2, mask=own2)
                    return jnp.logical_and(r, jnp.logical_not(own2)), it + 1

                lax.while_loop(_cond, _body, (rem, jnp.int32(0)))
            return carry
        lax.fori_loop(0, GPC, _g, 0, unroll=2)

    _start(0, sbufa, dbufa, sema)

    def _chunks(i, carry):
        ca = 2 * i
        _start(ca + 1, sbufb, dbufb, semb)
        _wait(sbufa, dbufa, sema)
        _groups(sbufa, dbufa)

        @pl.when(ca + 2 < NCH)
        def _prefetch():
            _start(ca + 2, sbufa, dbufa, sema)
        _wait(sbufb, dbufb, semb)
        _groups(sbufb, dbufb)
        return carry
    lax.fori_loop(0, NCH // 2, _chunks, 0)

    def _fix(i, carry):
        sl = pl.ds(i * LANES, LANES)
        for f in range(FPW):
            a = acs[f][sl]
            acs[f][sl] = jnp.where(a < 0, hcs[f][sl], a)
        return carry
    lax.fori_loop(0, N // LANES, _fix, 0, unroll=4)

    for f in range(FPW):
        pltpu.sync_copy(acs[f], out.at[wid, f])


@functools.partial(
    pl.kernel,
    mesh=plsc.VectorSubcoreMesh(core_axis_name="c", subcore_axis_name="s"),
    out_type=jax.ShapeDtypeStruct((NW, FPW, N), jnp.float32),
    compiler_params=pltpu.CompilerParams(needs_layout_passes=False),
    scratch_types=[
        pltpu.VMEM((N,), jnp.float32),
        pltpu.VMEM((N,), jnp.float32),
        pltpu.VMEM((N,), jnp.float32),
        pltpu.VMEM((N,), jnp.float32),
        pltpu.VMEM((N,), jnp.float32),
        pltpu.VMEM((N,), jnp.float32),
        pltpu.VMEM((N,), jnp.float32),
        pltpu.VMEM((N,), jnp.float32),
        pltpu.VMEM((N,), jnp.int32),
        pltpu.VMEM((CH,), jnp.int32),
        pltpu.VMEM((CH,), jnp.int32),
        pltpu.VMEM((CH,), jnp.int32),
        pltpu.VMEM((CH,), jnp.int32),
        pltpu.SemaphoreType.DMA,
        pltpu.SemaphoreType.DMA,
    ],
)
def _gcn_sc(hc, src, dst, out, hc0, hc1, hc2, hc3, ac0, ac1, ac2, ac3, scr,
            sbufa, dbufa, sbufb, dbufb, sema, semb):
    _gcn_sc_body(hc, src, dst, out, hc0, hc1, hc2, hc3,
                 ac0, ac1, ac2, ac3, scr,
                 sbufa, dbufa, sbufb, dbufb, sema, semb)


def _to_fmajor(h):
    return h.T.reshape(NW, FPW, N)


def _from_fmajor(a):
    return a.reshape(D, N).T


def kernel(x, edge_index, W0, b0, g0, be0, W1, b1, g1, be1):
    ei = edge_index.astype(jnp.int32)
    src = ei[0]
    dst = ei[1]

    h1 = _mlp1(x, W0, b0, g0, be0)
    a1 = _from_fmajor(_gcn_sc(_to_fmajor(h1), src, dst))
    h2 = _mlp2(h1, a1, W1[:D], W1[D:], b1, g1, be1)
    a2 = _from_fmajor(_gcn_sc(_to_fmajor(h2), src, dst))
    return jnp.concatenate([h2, a2], axis=1)


# speedup vs baseline: 2.2780x; 1.0254x over previous
import functools

import jax
import jax.numpy as jnp
from jax import lax
from jax.experimental import pallas as pl
from jax.experimental.pallas import tpu as pltpu
from jax.experimental.pallas import tpu_sc as plsc

N = 10000
D = 128
E = 320000

NC = 2
NS = 16
NW = NC * NS
FPW = D // NW
LANES = 16
CH = 1600
NCH = E // CH
GPC = CH // LANES


def _mlp1_body(x_ref, w_ref, b_ref, g_ref, be_ref, o_ref):
    h = jnp.dot(x_ref[...], w_ref[...], preferred_element_type=jnp.float32,
                precision=lax.Precision.HIGHEST)
    h = h + b_ref[...]
    mu = jnp.mean(h, axis=-1, keepdims=True)
    var = jnp.mean(jnp.square(h - mu), axis=-1, keepdims=True)
    h = (h - mu) / jnp.sqrt(var + 1e-5) * g_ref[...] + be_ref[...]
    o_ref[...] = jnp.maximum(h, 0.0)


def _mlp2_body(h_ref, a_ref, wa_ref, wb_ref, b_ref, g_ref, be_ref, o_ref):
    h = jnp.dot(h_ref[...], wa_ref[...], preferred_element_type=jnp.float32,
                precision=lax.Precision.HIGHEST)
    h = h + jnp.dot(a_ref[...], wb_ref[...], preferred_element_type=jnp.float32,
                    precision=lax.Precision.HIGHEST)
    h = h + b_ref[...]
    mu = jnp.mean(h, axis=-1, keepdims=True)
    var = jnp.mean(jnp.square(h - mu), axis=-1, keepdims=True)
    h = (h - mu) / jnp.sqrt(var + 1e-5) * g_ref[...] + be_ref[...]
    o_ref[...] = jnp.maximum(h, 0.0)


_ROWS = 1000


def _mlp1(x, w, b, g, be):
    vec = pl.BlockSpec((1, D), lambda i: (0, 0))
    return pl.pallas_call(
        _mlp1_body,
        grid=(N // _ROWS,),
        in_specs=[
            pl.BlockSpec((_ROWS, D), lambda i: (i, 0)),
            pl.BlockSpec((D, D), lambda i: (0, 0)),
            vec, vec, vec,
        ],
        out_specs=pl.BlockSpec((_ROWS, D), lambda i: (i, 0)),
        out_shape=jax.ShapeDtypeStruct((N, D), jnp.float32),
    )(x, w, b.reshape(1, D), g.reshape(1, D), be.reshape(1, D))


def _mlp2(h, a, wa, wb, b, g, be):
    vec = pl.BlockSpec((1, D), lambda i: (0, 0))
    return pl.pallas_call(
        _mlp2_body,
        grid=(N // _ROWS,),
        in_specs=[
            pl.BlockSpec((_ROWS, D), lambda i: (i, 0)),
            pl.BlockSpec((_ROWS, D), lambda i: (i, 0)),
            pl.BlockSpec((D, D), lambda i: (0, 0)),
            pl.BlockSpec((D, D), lambda i: (0, 0)),
            vec, vec, vec,
        ],
        out_specs=pl.BlockSpec((_ROWS, D), lambda i: (i, 0)),
        out_shape=jax.ShapeDtypeStruct((N, D), jnp.float32),
    )(h, a, wa, wb, b.reshape(1, D), g.reshape(1, D), be.reshape(1, D))


def _gcn_sc_body(hc, src, dst, out, hc0, hc1, hc2, hc3,
                 ac0, ac1, ac2, ac3, scr0, scr1,
                 sbufa, dbufa, sbufb, dbufb, sema, semb):
    wid = lax.axis_index("s") * NC + lax.axis_index("c")
    lane = jnp.arange(LANES, dtype=jnp.int32)
    hcs = (hc0, hc1, hc2, hc3)
    acs = (ac0, ac1, ac2, ac3)

    for f in range(FPW):
        pltpu.sync_copy(hc.at[wid, f], hcs[f])

    def _init(i, carry):
        neg = jnp.full((LANES,), -1.0, jnp.float32)
        for f in range(FPW):
            acs[f][pl.ds(i * LANES, LANES)] = neg
        return carry
    lax.fori_loop(0, N // LANES, _init, 0, unroll=4)

    def _start(c, sbuf, dbuf, sem):
        pltpu.async_copy(src.at[pl.ds(c * CH, CH)], sbuf, sem)
        pltpu.async_copy(dst.at[pl.ds(c * CH, CH)], dbuf, sem)

    def _wait(sbuf, dbuf, sem):
        pltpu.make_async_copy(src.at[pl.ds(0, CH)], sbuf, sem).wait()
        pltpu.make_async_copy(dst.at[pl.ds(0, CH)], dbuf, sem).wait()

    def _one_group(g, scr):
        s = sbuf_cur[0][pl.ds(g * LANES, LANES)]
        d = sbuf_cur[1][pl.ds(g * LANES, LANES)]
        plsc.store_scatter(scr, [d], lane)
        rb = plsc.load_gather(scr, [d])
        own = rb == lane
        vs = []
        for f in range(FPW):
            v = plsc.load_gather(hcs[f], [s])
            cur = plsc.load_gather(acs[f], [d])
            m = jnp.maximum(cur, v)
            plsc.store_scatter(acs[f], [d], m, mask=own)
            vs.append(v)
        rem = jnp.logical_not(own)

        @pl.when(jnp.any(rem))
        def _fallback():
            def _cond(c2):
                r, it = c2
                return jnp.logical_and(jnp.any(r), it < LANES)

            def _body(c2):
                r, it = c2
                plsc.store_scatter(scr, [d], lane, mask=r)
                rb2 = plsc.load_gather(scr, [d])
                own2 = jnp.logical_and(rb2 == lane, r)
                for f in range(FPW):
                    cur2 = plsc.load_gather(acs[f], [d])
                    m2 = jnp.maximum(cur2, vs[f])
                    plsc.store_scatter(acs[f], [d], m2, mask=own2)
                return jnp.logical_and(r, jnp.logical_not(own2)), it + 1

            lax.while_loop(_cond, _body, (rem, jnp.int32(0)))

    sbuf_cur = [None, None]

    def _groups(sbuf, dbuf):
        sbuf_cur[0] = sbuf
        sbuf_cur[1] = dbuf

        def _pair(p, carry):
            _one_group(p * 2, scr0)
            _one_group(p * 2 + 1, scr1)
            return carry
        lax.fori_loop(0, GPC // 2, _pair, 0, unroll=2)

    _start(0, sbufa, dbufa, sema)

    def _chunks(i, carry):
        ca = 2 * i
        _start(ca + 1, sbufb, dbufb, semb)
        _wait(sbufa, dbufa, sema)
        _groups(sbufa, dbufa)

        @pl.when(ca + 2 < NCH)
        def _prefetch():
            _start(ca + 2, sbufa, dbufa, sema)
        _wait(sbufb, dbufb, semb)
        _groups(sbufb, dbufb)
        return carry
    lax.fori_loop(0, NCH // 2, _chunks, 0)

    def _fix(i, carry):
        sl = pl.ds(i * LANES, LANES)
        for f in range(FPW):
            a = acs[f][sl]
            acs[f][sl] = jnp.where(a < 0, hcs[f][sl], a)
        return carry
    lax.fori_loop(0, N // LANES, _fix, 0, unroll=4)

    for f in range(FPW):
        pltpu.sync_copy(acs[f], out.at[wid, f])


@functools.partial(
    pl.kernel,
    mesh=plsc.VectorSubcoreMesh(core_axis_name="c", subcore_axis_name="s"),
    out_type=jax.ShapeDtypeStruct((NW, FPW, N), jnp.float32),
    compiler_params=pltpu.CompilerParams(needs_layout_passes=False),
    scratch_types=[
        pltpu.VMEM((N,), jnp.float32),
        pltpu.VMEM((N,), jnp.float32),
        pltpu.VMEM((N,), jnp.float32),
        pltpu.VMEM((N,), jnp.float32),
        pltpu.VMEM((N,), jnp.float32),
        pltpu.VMEM((N,), jnp.float32),
        pltpu.VMEM((N,), jnp.float32),
        pltpu.VMEM((N,), jnp.float32),
        pltpu.VMEM((N,), jnp.int32),
        pltpu.VMEM((N,), jnp.int32),
        pltpu.VMEM((CH,), jnp.int32),
        pltpu.VMEM((CH,), jnp.int32),
        pltpu.VMEM((CH,), jnp.int32),
        pltpu.VMEM((CH,), jnp.int32),
        pltpu.SemaphoreType.DMA,
        pltpu.SemaphoreType.DMA,
    ],
)
def _gcn_sc(hc, src, dst, out, hc0, hc1, hc2, hc3, ac0, ac1, ac2, ac3,
            scr0, scr1, sbufa, dbufa, sbufb, dbufb, sema, semb):
    _gcn_sc_body(hc, src, dst, out, hc0, hc1, hc2, hc3,
                 ac0, ac1, ac2, ac3, scr0, scr1,
                 sbufa, dbufa, sbufb, dbufb, sema, semb)


def _to_fmajor(h):
    return h.T.reshape(NW, FPW, N)


def _from_fmajor(a):
    return a.reshape(D, N).T


def kernel(x, edge_index, W0, b0, g0, be0, W1, b1, g1, be1):
    ei = edge_index.astype(jnp.int32)
    src = ei[0]
    dst = ei[1]

    h1 = _mlp1(x, W0, b0, g0, be0)
    a1 = _from_fmajor(_gcn_sc(_to_fmajor(h1), src, dst))
    h2 = _mlp2(h1, a1, W1[:D], W1[D:], b1, g1, be1)
    a2 = _from_fmajor(_gcn_sc(_to_fmajor(h2), src, dst))
    return jnp.concatenate([h2, a2], axis=1)
